# baseline (speedup 1.0000x reference)
import functools

import jax
import jax.numpy as jnp
from jax import lax
from jax.experimental import pallas as pl
from jax.experimental.pallas import tpu as pltpu
from jax.experimental.pallas import tpu_sc as plsc

N = 10000
E = 320000
D_IN = 128
D_HID = 32
D_OUT = 4

NP = 10240
NC = 2
NS = 16
NW = NC * NS
EPW = E // NW
CHUNK = 128
K = 8
NR = 10
CPW = K * NR
FULL = EPW // CHUNK
TAIL = EPW - FULL * CHUNK
ROWS_PER_SUB = NP // NS

_mesh = functools.partial(
    plsc.VectorSubcoreMesh,
    core_axis_name="c", subcore_axis_name="s",
    num_cores=NC, num_subcores=NS,
)

_SC_PARAMS = pltpu.CompilerParams(use_tc_tiling_on_sc=False)


def _prefill_dummy(idxv):
    lane = lax.iota(jnp.int32, 16)
    for g in range(TAIL // 16, CHUNK // 16):
        idxv[FULL, pl.ds(g * 16, 16)] = N + g * 16 + lane
    for g in range(CHUNK // 16):
        idxv[FULL + 1, pl.ds(g * 16, 16)] = N + g * 16 + lane


def _load_chunks(idx_hbm, wid, idxv, sem):
    def fire(c, carry):
        pltpu.async_copy(idx_hbm.at[wid, pl.ds(c * CHUNK, CHUNK)],
                         idxv.at[c], sem)
        return carry

    lax.fori_loop(0, FULL, fire, 0)
    pltpu.async_copy(idx_hbm.at[wid, pl.ds(FULL * CHUNK, TAIL)],
                     idxv.at[FULL, pl.ds(0, TAIL)], sem)

    def drain(c, carry):
        pltpu.make_async_copy(idx_hbm.at[wid, pl.ds(c * CHUNK, CHUNK)],
                              idxv.at[c], sem).wait()
        return carry

    lax.fori_loop(0, FULL, drain, 0)
    pltpu.make_async_copy(idx_hbm.at[wid, pl.ds(FULL * CHUNK, TAIL)],
                          idxv.at[FULL, pl.ds(0, TAIL)], sem).wait()


def _deg_kernel(dst_hbm, zeros_hbm, degp_hbm, dstv, ones, semi, dsh):
    core = lax.axis_index("c")
    sub = lax.axis_index("s")
    wid = core * NS + sub
    rows = pl.ds(sub * ROWS_PER_SUB, ROWS_PER_SUB)
    pltpu.sync_copy(zeros_hbm.at[rows], dsh.at[rows])
    for k in range(CHUNK // 16):
        ones[pl.ds(k * 16, 16)] = jnp.ones((16,), jnp.float32)
    _prefill_dummy(dstv)
    _load_chunks(dst_hbm, wid, dstv, semi)
    plsc.subcore_barrier()

    def body(c, carry):
        pltpu.sync_copy(ones, dsh.at[dstv.at[c]], add=True)
        return carry

    lax.fori_loop(0, CPW, body, 0)
    plsc.subcore_barrier()
    pltpu.sync_copy(dsh.at[rows], degp_hbm.at[core, rows])


def _sc_degree(dst2, zeros1):
    return pl.kernel(
        _deg_kernel,
        out_type=jax.ShapeDtypeStruct((NC, NP), jnp.float32),
        mesh=_mesh(),
        scratch_types=[
            pltpu.VMEM((CPW, CHUNK), jnp.int32),
            pltpu.VMEM((CHUNK,), jnp.float32),
            pltpu.SemaphoreType.DMA,
            pltpu.VMEM_SHARED((NP,), jnp.float32),
        ],
        compiler_params=_SC_PARAMS,
    )(dst2, zeros1)


def _msg_kernel(y_hbm, src_hbm, dst_hbm, zeros_hbm, sp_hbm,
                srcv, dstv, buf, semi, semg_a, semg_b, sems_a, sems_b, ssh):
    core = lax.axis_index("c")
    sub = lax.axis_index("s")
    wid = core * NS + sub
    rows = pl.ds(sub * ROWS_PER_SUB, ROWS_PER_SUB)
    pltpu.sync_copy(zeros_hbm.at[rows], ssh.at[rows])
    _prefill_dummy(srcv)
    _prefill_dummy(dstv)
    _load_chunks(src_hbm, wid, srcv, semi)
    _load_chunks(dst_hbm, wid, dstv, semi)
    plsc.subcore_barrier()

    def fire_gathers(round_idx, group, semg):
        for j in range(K):
            pltpu.async_copy(
                y_hbm.at[srcv.at[round_idx * K + j]], buf.at[group * K + j],
                semg)

    def do_round(round_idx, group, semg, sems):
        scat = []
        for j in range(K):
            pltpu.make_async_copy(
                y_hbm.at[srcv.at[round_idx * K + j]], buf.at[group * K + j],
                semg).wait()
        for j in range(K):
            scat.append(pltpu.async_copy(
                buf.at[group * K + j], ssh.at[dstv.at[round_idx * K + j]],
                sems, add=True))
        for d in scat:
            d.wait()

    fire_gathers(0, 0, semg_a)
    fire_gathers(1, 1, semg_b)

    def pair(r2, carry):
        ra = 2 * r2
        do_round(ra, 0, semg_a, sems_a)
        fire_gathers(ra + 2, 0, semg_a)
        do_round(ra + 1, 1, semg_b, sems_b)
        fire_gathers(ra + 3, 1, semg_b)
        return carry

    lax.fori_loop(0, NR // 2 - 1, pair, 0)
    do_round(NR - 2, 0, semg_a, sems_a)
    do_round(NR - 1, 1, semg_b, sems_b)

    plsc.subcore_barrier()
    pltpu.sync_copy(ssh.at[rows], sp_hbm.at[core, rows])


def _sc_aggregate(y, src2, dst2, d):
    zeros2 = jnp.zeros((NP, d), jnp.float32)
    return pl.kernel(
        _msg_kernel,
        out_type=jax.ShapeDtypeStruct((NC, NP, d), jnp.float32),
        mesh=_mesh(),
        scratch_types=[
            pltpu.VMEM((CPW, CHUNK), jnp.int32),
            pltpu.VMEM((CPW, CHUNK), jnp.int32),
            pltpu.VMEM((2 * K, CHUNK, d), jnp.float32),
            pltpu.SemaphoreType.DMA,
            pltpu.SemaphoreType.DMA,
            pltpu.SemaphoreType.DMA,
            pltpu.SemaphoreType.DMA,
            pltpu.SemaphoreType.DMA,
            pltpu.VMEM_SHARED((NP, d), jnp.float32),
        ],
        compiler_params=_SC_PARAMS,
    )(y, src2, dst2, zeros2)


def _tcmm_body(x_ref, w_ref, xw_ref):
    xw_ref[...] = jnp.dot(x_ref[...], w_ref[...],
                          preferred_element_type=jnp.float32)


def _tcmm(xp, W1):
    return pl.pallas_call(
        _tcmm_body,
        out_shape=jax.ShapeDtypeStruct((NP, D_HID), jnp.float32),
    )(xp, W1)


def _tcscale_body(xw_ref, dp_ref, y1_ref, dinv_ref):
    deg = dp_ref[0, :] + dp_ref[1, :] + 1.0
    dinv = lax.rsqrt(deg)[:, None]
    dinv_ref[...] = dinv
    y1_ref[...] = xw_ref[...] * dinv


def _tcscale(xw, degp):
    return pl.pallas_call(
        _tcscale_body,
        out_shape=[
            jax.ShapeDtypeStruct((NP, D_HID), jnp.float32),
            jax.ShapeDtypeStruct((NP, 1), jnp.float32),
        ],
    )(xw, degp)


def _tc2_body(p_ref, y1_ref, dinv_ref, b1_ref, w2_ref, y2_ref):
    dinv = dinv_ref[...]
    s = p_ref[0] + p_ref[1] + y1_ref[...]
    h = jnp.maximum(dinv * s + b1_ref[...], 0.0)
    y2_ref[...] = jnp.dot(h, w2_ref[...], preferred_element_type=jnp.float32) * dinv


def _tc2(sp1, y1, dinv, b1r, W2p):
    return pl.pallas_call(
        _tc2_body,
        out_shape=jax.ShapeDtypeStruct((NP, 8), jnp.float32),
    )(sp1, y1, dinv, b1r, W2p)


def _tc3_body(q_ref, y2_ref, dinv_ref, b2_ref, out_ref):
    s = q_ref[0] + q_ref[1] + y2_ref[...]
    out_ref[...] = (dinv_ref[...] * s + b2_ref[...])[:N, :D_OUT]


def _tc3(sp2, y2, dinv, b2r):
    return pl.pallas_call(
        _tc3_body,
        out_shape=jax.ShapeDtypeStruct((N, D_OUT), jnp.float32),
    )(sp2, y2, dinv, b2r)


def kernel(x, edge_index, W1, b1, W2, b2):
    ei = edge_index.astype(jnp.int32)
    dst2 = lax.optimization_barrier(ei[1].reshape(NW, EPW))
    src2 = ei[0].reshape(NW, EPW)
    xp = jnp.pad(x, ((0, NP - N), (0, 0)))
    W2p = jnp.pad(W2, ((0, 0), (0, 8 - D_OUT)))
    b1r = b1[None, :]
    b2r = jnp.pad(b2, (0, 8 - D_OUT))[None, :]
    zeros1 = jnp.zeros((NP,), jnp.float32)

    degp = _sc_degree(dst2, zeros1)
    xw = _tcmm(xp, W1)
    y1, dinv = _tcscale(xw, degp)
    sp1 = _sc_aggregate(y1, src2, dst2, D_HID)
    y2 = _tc2(sp1, y1, dinv, b1r, W2p)
    sp2 = _sc_aggregate(y2, src2, dst2, 8)
    return _tc3(sp2, y2, dinv, b2r)


# speedup vs baseline: 1.2363x; 1.2363x over previous
import functools

import jax
import jax.numpy as jnp
from jax import lax
from jax.experimental import pallas as pl
from jax.experimental.pallas import tpu as pltpu
from jax.experimental.pallas import tpu_sc as plsc

N = 10000
E = 320000
D_IN = 128
D_HID = 32
D_OUT = 4

NP = 10240
NC = 2
NS = 16
NW = NC * NS
EPW = E // NW
CHUNK = 128
K = 8
NR = 10
CPW = K * NR
FULL = EPW // CHUNK
TAIL = EPW - FULL * CHUNK
ROWS_PER_SUB = NP // NS

_mesh = functools.partial(
    plsc.VectorSubcoreMesh,
    core_axis_name="c", subcore_axis_name="s",
    num_cores=NC, num_subcores=NS,
)

_SC_PARAMS = pltpu.CompilerParams(use_tc_tiling_on_sc=False)


def _prefill_dummy(idxv):
    lane = lax.iota(jnp.int32, 16)
    for g in range(TAIL // 16, CHUNK // 16):
        idxv[FULL, pl.ds(g * 16, 16)] = N + g * 16 + lane
    for g in range(CHUNK // 16):
        idxv[FULL + 1, pl.ds(g * 16, 16)] = N + g * 16 + lane


def _load_chunks(idx_hbm, wid, idxv, sem):
    def fire(c, carry):
        pltpu.async_copy(idx_hbm.at[wid, pl.ds(c * CHUNK, CHUNK)],
                         idxv.at[c], sem)
        return carry

    lax.fori_loop(0, FULL, fire, 0)
    pltpu.async_copy(idx_hbm.at[wid, pl.ds(FULL * CHUNK, TAIL)],
                     idxv.at[FULL, pl.ds(0, TAIL)], sem)

    def drain(c, carry):
        pltpu.make_async_copy(idx_hbm.at[wid, pl.ds(c * CHUNK, CHUNK)],
                              idxv.at[c], sem).wait()
        return carry

    lax.fori_loop(0, FULL, drain, 0)
    pltpu.make_async_copy(idx_hbm.at[wid, pl.ds(FULL * CHUNK, TAIL)],
                          idxv.at[FULL, pl.ds(0, TAIL)], sem).wait()


def _deg_kernel(dst_hbm, zeros_hbm, degp_hbm, dstv, ones, semi, dsh):
    core = lax.axis_index("c")
    sub = lax.axis_index("s")
    wid = core * NS + sub
    rows = pl.ds(sub * ROWS_PER_SUB, ROWS_PER_SUB)
    pltpu.sync_copy(zeros_hbm.at[rows], dsh.at[rows])
    for k in range(CHUNK // 16):
        ones[pl.ds(k * 16, 16)] = jnp.ones((16,), jnp.float32)
    _prefill_dummy(dstv)
    _load_chunks(dst_hbm, wid, dstv, semi)
    plsc.subcore_barrier()

    def body(c, carry):
        pltpu.sync_copy(ones, dsh.at[dstv.at[c]], add=True)
        return carry

    lax.fori_loop(0, CPW, body, 0)
    plsc.subcore_barrier()
    pltpu.sync_copy(dsh.at[rows], degp_hbm.at[core, rows])


def _sc_degree(dst2, zeros1):
    return pl.kernel(
        _deg_kernel,
        out_type=jax.ShapeDtypeStruct((NC, NP), jnp.float32),
        mesh=_mesh(),
        scratch_types=[
            pltpu.VMEM((CPW, CHUNK), jnp.int32),
            pltpu.VMEM((CHUNK,), jnp.float32),
            pltpu.SemaphoreType.DMA,
            pltpu.VMEM_SHARED((NP,), jnp.float32),
        ],
        compiler_params=_SC_PARAMS,
    )(dst2, zeros1)


def _msg_kernel(y_hbm, src_hbm, dst_hbm, zeros_hbm, sp_hbm,
                srcv, dstv, buf, semi, semg_a, semg_b, sems_a, sems_b, ssh):
    core = lax.axis_index("c")
    sub = lax.axis_index("s")
    wid = core * NS + sub
    rows = pl.ds(sub * ROWS_PER_SUB, ROWS_PER_SUB)
    pltpu.sync_copy(zeros_hbm.at[rows], ssh.at[rows])
    _prefill_dummy(srcv)
    _prefill_dummy(dstv)
    _load_chunks(src_hbm, wid, srcv, semi)
    _load_chunks(dst_hbm, wid, dstv, semi)
    plsc.subcore_barrier()

    def fire_gathers(round_idx, group, semg):
        for j in range(K):
            pltpu.async_copy(
                y_hbm.at[srcv.at[round_idx * K + j]], buf.at[group * K + j],
                semg)

    def do_round(round_idx, group, semg, sems):
        scat = []
        for j in range(K):
            pltpu.make_async_copy(
                y_hbm.at[srcv.at[round_idx * K + j]], buf.at[group * K + j],
                semg).wait()
        for j in range(K):
            scat.append(pltpu.async_copy(
                buf.at[group * K + j], ssh.at[dstv.at[round_idx * K + j]],
                sems, add=True))
        for d in scat:
            d.wait()

    fire_gathers(0, 0, semg_a)
    fire_gathers(1, 1, semg_b)

    def pair(r2, carry):
        ra = 2 * r2
        do_round(ra, 0, semg_a, sems_a)
        fire_gathers(ra + 2, 0, semg_a)
        do_round(ra + 1, 1, semg_b, sems_b)
        fire_gathers(ra + 3, 1, semg_b)
        return carry

    lax.fori_loop(0, NR // 2 - 1, pair, 0)
    do_round(NR - 2, 0, semg_a, sems_a)
    do_round(NR - 1, 1, semg_b, sems_b)

    plsc.subcore_barrier()
    pltpu.sync_copy(ssh.at[rows], sp_hbm.at[core, rows])


def _sc_aggregate(y, src2, dst2, d):
    zeros2 = jnp.zeros((NP, d), jnp.float32)
    return pl.kernel(
        _msg_kernel,
        out_type=jax.ShapeDtypeStruct((NC, NP, d), jnp.float32),
        mesh=_mesh(),
        scratch_types=[
            pltpu.VMEM((CPW, CHUNK), jnp.int32),
            pltpu.VMEM((CPW, CHUNK), jnp.int32),
            pltpu.VMEM((2 * K, CHUNK, d), jnp.float32),
            pltpu.SemaphoreType.DMA,
            pltpu.SemaphoreType.DMA,
            pltpu.SemaphoreType.DMA,
            pltpu.SemaphoreType.DMA,
            pltpu.SemaphoreType.DMA,
            pltpu.VMEM_SHARED((NP, d), jnp.float32),
        ],
        compiler_params=_SC_PARAMS,
    )(y, src2, dst2, zeros2)


R1 = NP * D_HID // 512
R8 = NP * 8 // 128


def _tcdinv_body(dp_ref, dinv_ref):
    dinv_ref[...] = lax.rsqrt(dp_ref[0] + dp_ref[1] + 1.0)


def _tcdinv(degp3):
    return pl.pallas_call(
        _tcdinv_body,
        out_shape=jax.ShapeDtypeStruct((NP // 128, 128), jnp.float32),
    )(degp3)


def _tcmm_body(x_ref, w_ref, xw_ref):
    xw_ref[...] = jnp.dot(x_ref[...], w_ref[...],
                          preferred_element_type=jnp.float32)


def _tcmm(xp2048, W1blk):
    return pl.pallas_call(
        _tcmm_body,
        out_shape=jax.ShapeDtypeStruct((R1, 512), jnp.float32),
    )(xp2048, W1blk)


def _tc2_body(p_ref, y1_ref, de_ref, de8_ref, b1_ref, w2_ref, y2_ref):
    s = p_ref[0] + p_ref[1] + y1_ref[...]
    h = jnp.maximum(de_ref[...] * s + b1_ref[...], 0.0)
    y2_ref[...] = jnp.dot(h, w2_ref[...],
                          preferred_element_type=jnp.float32) * de8_ref[...]


def _tc2(sp1f, y1T, dinvE, dinvE8, b1t, W2blk):
    return pl.pallas_call(
        _tc2_body,
        out_shape=jax.ShapeDtypeStruct((R8, 128), jnp.float32),
    )(sp1f, y1T, dinvE, dinvE8, b1t, W2blk)


def _tc3_body(q_ref, y2_ref, de8_ref, b2_ref, out_ref):
    s = q_ref[0] + q_ref[1] + y2_ref[...]
    out_ref[...] = de8_ref[...] * s + b2_ref[...]


def _tc3(sp2f, y2T, dinvE8, b2t):
    return pl.pallas_call(
        _tc3_body,
        out_shape=jax.ShapeDtypeStruct((R8, 128), jnp.float32),
    )(sp2f, y2T, dinvE8, b2t)


def kernel(x, edge_index, W1, b1, W2, b2):
    ei = edge_index.astype(jnp.int32)
    src2 = ei[0].reshape(NW, EPW)
    dst2 = ei[1].reshape(NW, EPW)
    xp = jnp.pad(x, ((0, NP - N), (0, 0)))
    W2p = jnp.pad(W2, ((0, 0), (0, 8 - D_OUT)))
    eye = jnp.eye(16, dtype=jnp.float32)
    W1blk = (eye[:, None, :, None] * W1[None, :, None, :]).reshape(2048, 512)
    W2blk = (eye[:, None, :, None] * W2p[None, :, None, :]).reshape(512, 128)
    b1t = jnp.tile(b1, 16)[None, :]
    b2t = jnp.tile(jnp.pad(b2, (0, 8 - D_OUT)), 16)[None, :]
    zeros1 = jnp.zeros((NP,), jnp.float32)

    degp = _sc_degree(dst2, zeros1)
    xw512 = _tcmm(xp.reshape(R1, 2048), W1blk)
    dinv_col = _tcdinv(degp.reshape(NC, NP // 128, 128)).reshape(NP, 1)
    dinvE = jnp.broadcast_to(dinv_col, (NP, D_HID)).reshape(R1, 512)
    dinvE8 = jnp.broadcast_to(dinv_col, (NP, 8)).reshape(R8, 128)
    y1T = xw512 * dinvE
    sp1 = _sc_aggregate(y1T.reshape(NP, D_HID), src2, dst2, D_HID)
    y2T = _tc2(sp1.reshape(NC, R1, 512), y1T, dinvE, dinvE8, b1t, W2blk)
    sp2 = _sc_aggregate(y2T.reshape(NP, 8), src2, dst2, 8)
    res = _tc3(sp2.reshape(NC, R8, 128), y2T, dinvE8, b2t)
    return res.reshape(NP, 8)[:N, :D_OUT]
